# baseline (speedup 1.0000x reference)
import functools

import jax
import jax.numpy as jnp
from jax import lax
from jax.experimental import pallas as pl
from jax.experimental.pallas import tpu as pltpu
from jax.experimental.pallas import tpu_sc as plsc

L = 200
D = 64
NBH = 32
BW = 128


def _sc_gather(table, idx3):
    info = plsc.get_sparse_core_info()
    nc = info.num_cores
    mesh = plsc.VectorSubcoreMesh(core_axis_name="c", subcore_axis_name="s")

    @functools.partial(
        pl.kernel,
        out_type=jax.ShapeDtypeStruct((L, 8, NBH, 8 * BW), jnp.float32),
        mesh=mesh,
        scratch_types=[
            pltpu.VMEM((L, BW), jnp.int32),
            pltpu.VMEM((2, BW, D), jnp.float32),
            pltpu.VMEM((2, 8, 8 * BW), jnp.float32),
            pltpu.SemaphoreType.DMA,
            pltpu.SemaphoreType.DMA,
            pltpu.SemaphoreType.DMA,
            pltpu.SemaphoreType.DMA,
        ],
        compiler_params=pltpu.CompilerParams(
            use_tc_tiling_on_sc=False, needs_layout_passes=False
        ),
    )
    def k(table_hbm, idx_hbm, out_hbm, idx_v, rows_v, blk_v,
          sem_g0, sem_g1, sem_o0, sem_o1):
        wid = lax.axis_index("s") * nc + lax.axis_index("c")
        sem_g = (sem_g0, sem_g1)
        sem_o = (sem_o0, sem_o1)

        def gather(l, p):
            return pltpu.make_async_copy(
                table_hbm.at[idx_v.at[l]], rows_v.at[p], sem_g[p]
            )

        def store(l, p):
            return pltpu.make_async_copy(
                blk_v.at[p], out_hbm.at[l, :, wid], sem_o[p]
            )

        pltpu.sync_copy(idx_hbm.at[:, wid], idx_v)

        gather(0, 0).start()
        gather(1, 1).start()

        lane = jnp.arange(16, dtype=jnp.int32)
        rows = [lane + (16 * j) for j in range(8)]

        def transpose(p):
            @plsc.parallel_loop(0, 16, unroll=1)
            def tr(k):
                rot = (lane + k) & 15
                rhi = rot >> 3
                clow = ((rot & 7) << 7) + lane
                for m in range(4):
                    lcol = rot + (16 * m)
                    rowm = rhi + (2 * m)
                    for j in range(8):
                        vec = plsc.load_gather(
                            rows_v.at[p], [rows[j], lcol]
                        )
                        plsc.store_scatter(
                            blk_v.at[p], [rowm, clow + (16 * j)], vec
                        )

        def body(i, _):
            for p in (0, 1):
                l = 2 * i + p
                gather(l, p).wait()

                @pl.when(i >= 1)
                def _wait_prev_store():
                    store(l - 2, p).wait()

                transpose(p)
                store(l, p).start()

                @pl.when(l + 2 < L)
                def _next_gather():
                    gather(l + 2, p).start()
            return _

        lax.fori_loop(0, L // 2, body, 0)
        store(L - 2, 0).wait()
        store(L - 1, 1).wait()

    return k(table, idx3)


def kernel(code, static_mask, numeric_value, time_delta_days,
           numeric_value_mask, mask, table):
    B, _ = code.shape
    idx3 = code.astype(jnp.int32).T.reshape(L, NBH, BW)
    x = _sc_gather(table, idx3)
    x = x.reshape(L, 8, NBH, 8, BW)
    y = jnp.transpose(x, (2, 4, 0, 1, 3))
    return y.reshape(B, L, D)


# speedup vs baseline: 1.4308x; 1.4308x over previous
import functools

import jax
import jax.numpy as jnp
from jax import lax
from jax.experimental import pallas as pl
from jax.experimental.pallas import tpu as pltpu
from jax.experimental.pallas import tpu_sc as plsc

L = 200
D = 64
NBH = 32
BW = 128


def _sc_gather(table, idx3):
    info = plsc.get_sparse_core_info()
    nc = info.num_cores
    mesh = plsc.VectorSubcoreMesh(core_axis_name="c", subcore_axis_name="s")

    @functools.partial(
        pl.kernel,
        out_type=jax.ShapeDtypeStruct((L, 8, NBH, 8 * BW), jnp.float32),
        mesh=mesh,
        scratch_types=[
            pltpu.VMEM((L, BW), jnp.int32),
            pltpu.VMEM((2, BW, D), jnp.float32),
            pltpu.VMEM((2, 8, 8 * BW), jnp.float32),
            pltpu.SemaphoreType.DMA,
            pltpu.SemaphoreType.DMA,
            pltpu.SemaphoreType.DMA,
            pltpu.SemaphoreType.DMA,
        ],
        compiler_params=pltpu.CompilerParams(
            use_tc_tiling_on_sc=False, needs_layout_passes=False
        ),
    )
    def k(table_hbm, idx_hbm, out_hbm, idx_v, rows_v, blk_v,
          sem_g0, sem_g1, sem_o0, sem_o1):
        wid = lax.axis_index("s") * nc + lax.axis_index("c")
        sem_g = (sem_g0, sem_g1)
        sem_o = (sem_o0, sem_o1)

        def gather(l, p):
            return pltpu.make_async_copy(
                table_hbm.at[idx_v.at[l]], rows_v.at[p], sem_g[p]
            )

        def store(l, p):
            return pltpu.make_async_copy(
                blk_v.at[p], out_hbm.at[l, :, wid], sem_o[p]
            )

        pltpu.sync_copy(idx_hbm.at[:, wid], idx_v)

        gather(0, 0).start()
        gather(1, 1).start()

        lane = jnp.arange(16, dtype=jnp.int32)
        rows = [lane + (16 * j) for j in range(8)]

        def transpose(p):
            @plsc.parallel_loop(0, 16, unroll=2)
            def tr(k):
                rot = (lane + k) & 15
                rhi = rot >> 3
                clow = ((rot & 7) << 7) + lane
                for m in range(4):
                    lcol = rot + (16 * m)
                    rowm = rhi + (2 * m)
                    for j in range(8):
                        vec = plsc.load_gather(
                            rows_v.at[p], [rows[j], lcol]
                        )
                        plsc.store_scatter(
                            blk_v.at[p], [rowm, clow + (16 * j)], vec
                        )

        def body(i, _):
            for p in (0, 1):
                l = 2 * i + p
                gather(l, p).wait()

                @pl.when(i >= 1)
                def _wait_prev_store():
                    store(l - 2, p).wait()

                transpose(p)
                store(l, p).start()

                @pl.when(l + 2 < L)
                def _next_gather():
                    gather(l + 2, p).start()
            return _

        lax.fori_loop(0, L // 2, body, 0)
        store(L - 2, 0).wait()
        store(L - 1, 1).wait()

    return k(table, idx3)


def kernel(code, static_mask, numeric_value, time_delta_days,
           numeric_value_mask, mask, table):
    B, _ = code.shape
    idx3 = code.astype(jnp.int32).T.reshape(L, NBH, BW)
    x = _sc_gather(table, idx3)
    x = x.reshape(L, 8, NBH, 8, BW)
    y = jnp.transpose(x, (2, 4, 0, 1, 3))
    return y.reshape(B, L, D)
